# baseline (speedup 1.0000x reference)
from math import comb

import jax
import jax.numpy as jnp
from jax import lax
from jax.experimental import pallas as pl
from jax.experimental.pallas import tpu as pltpu
from jax.experimental.pallas import tpu_sc as plsc

NUM_CORES = 2
NUM_SUBCORES = 16
NUM_WORKERS = NUM_CORES * NUM_SUBCORES
LANES = 16
LUT_N = 512
CHUNK = 16384
VPI = 2
UNROLL = 8


def _build_lut(control_points, u, dtype):
    t = jnp.linspace(-1.0, 1.0, LUT_N, dtype=dtype)
    cpy = control_points[:, 1]
    n = control_points.shape[0] - 1
    bern = jnp.stack(
        [comb(n, k) * t**k * (1.0 - t) ** (n - k) for k in range(n + 1)], axis=0
    )
    fp = jnp.clip(cpy @ bern, -1.0, 1.0)
    slope = (fp[1:] - fp[:-1]) / (t[1:] - t[:-1])
    a_seg = fp[:-1] - slope * t[:-1]
    a = jnp.concatenate([a_seg[:1], a_seg])
    b = jnp.concatenate([slope[:1], slope])
    sign = jnp.where(u[0] > 0.5, 1.0, -1.0).astype(dtype)
    return jnp.stack([a * sign, b * sign])


def _sc_body(img_hbm, lut_hbm, out_hbm, xbuf, obuf, abuf, bbuf, isems, osems):
    pixels = img_hbm.shape[0] // NUM_WORKERS
    imgs_per_w = 2
    p = pixels // imgs_per_w
    nchunk = p // CHUNK

    cid = lax.axis_index("c")
    sid = lax.axis_index("s")
    wid = sid * NUM_CORES + cid

    pltpu.sync_copy(lut_hbm.at[0], abuf)
    pltpu.sync_copy(lut_hbm.at[1], bbuf)

    def interp(x):
        idx = jnp.clip((x * 255.5 + 256.5).astype(jnp.int32), 1, LUT_N - 1)
        av = plsc.load_gather(abuf, [idx])
        bv = plsc.load_gather(bbuf, [idx])
        return av + bv * x

    def start_in(base, c, b):
        pltpu.async_copy(
            img_hbm.at[pl.ds(base + c * CHUNK, CHUNK)],
            xbuf.at[pl.ds(b * CHUNK, CHUNK)],
            isems.at[b],
        )

    def wait_in(base, b):
        pltpu.make_async_copy(
            img_hbm.at[pl.ds(base, CHUNK)],
            xbuf.at[pl.ds(b * CHUNK, CHUNK)],
            isems.at[b],
        ).wait()

    def wait_out(base, b):
        pltpu.make_async_copy(
            obuf.at[pl.ds(b * CHUNK, CHUNK)],
            out_hbm.at[pl.ds(base, CHUNK)],
            osems.at[b],
        ).wait()

    for im in range(imgs_per_w):
        base = (wid * imgs_per_w + im) * p

        start_in(base, 0, 0)
        acc0 = (
            jnp.full((LANES,), jnp.inf, jnp.float32),
            jnp.full((LANES,), -jnp.inf, jnp.float32),
        ) * VPI

        def chunk1(c, acc):
            b = c % 2
            boff = b * CHUNK

            @pl.when(c + 1 < nchunk)
            def _():
                start_in(base, c + 1, 1 - b)

            wait_in(base, b)

            def step1(i, carry):
                out = []
                for k in range(VPI):
                    cmn, cmx = carry[2 * k], carry[2 * k + 1]
                    v = interp(xbuf[pl.ds(boff + i + k * LANES, LANES)])
                    out += [jnp.minimum(cmn, v), jnp.maximum(cmx, v)]
                return tuple(out)

            return plsc.parallel_loop(
                0, CHUNK, VPI * LANES, unroll=UNROLL, carry=acc
            )(step1)

        acc = lax.fori_loop(0, nchunk, chunk1, acc0)
        mn = jnp.minimum(acc[0], *[acc[2 * k] for k in range(1, VPI)])
        mx = jnp.maximum(acc[1], *[acc[2 * k + 1] for k in range(1, VPI)])
        mnv = jnp.full((LANES,), jnp.min(mn), jnp.float32)
        mxv = jnp.full((LANES,), jnp.max(mx), jnp.float32)
        scv = 2.0 / (mxv - mnv)
        ofv = -mnv * scv - 1.0

        start_in(base, 0, 0)

        def chunk2(c, carry):
            b = c % 2
            boff = b * CHUNK

            @pl.when(c + 1 < nchunk)
            def _():
                start_in(base, c + 1, 1 - b)

            wait_in(base, b)

            @pl.when(c >= 2)
            def _():
                wait_out(base, b)

            def step2(i, icarry):
                for k in range(VPI):
                    x = xbuf[pl.ds(boff + i + k * LANES, LANES)]
                    obuf[pl.ds(boff + i + k * LANES, LANES)] = (
                        interp(x) * scv + ofv
                    )
                return icarry

            plsc.parallel_loop(
                0, CHUNK, VPI * LANES, unroll=UNROLL, carry=jnp.int32(0)
            )(step2)
            pltpu.async_copy(
                obuf.at[pl.ds(boff, CHUNK)],
                out_hbm.at[pl.ds(base + c * CHUNK, CHUNK)],
                osems.at[b],
            )
            return carry

        lax.fori_loop(0, nchunk, chunk2, jnp.int32(0))
        wait_out(base, 0)
        wait_out(base, 1)


def kernel(image, control_points, u):
    shape = image.shape
    total = image.size
    lut = _build_lut(control_points, u, image.dtype)
    img_flat = image.reshape(total)

    mesh = plsc.VectorSubcoreMesh(
        core_axis_name="c",
        subcore_axis_name="s",
        num_cores=NUM_CORES,
        num_subcores=NUM_SUBCORES,
    )
    out = pl.kernel(
        _sc_body,
        out_type=jax.ShapeDtypeStruct((total,), jnp.float32),
        mesh=mesh,
        compiler_params=pltpu.CompilerParams(
            use_tc_tiling_on_sc=False, needs_layout_passes=False
        ),
        scratch_types=[
            pltpu.VMEM((2 * CHUNK,), jnp.float32),
            pltpu.VMEM((2 * CHUNK,), jnp.float32),
            pltpu.VMEM((LUT_N,), jnp.float32),
            pltpu.VMEM((LUT_N,), jnp.float32),
            pltpu.SemaphoreType.DMA((2,)),
            pltpu.SemaphoreType.DMA((2,)),
        ],
    )(img_flat, lut)
    return out.reshape(shape)


# speedup vs baseline: 1.4176x; 1.4176x over previous
from math import comb

import jax
import jax.numpy as jnp
from jax import lax
from jax.experimental import pallas as pl
from jax.experimental.pallas import tpu as pltpu
from jax.experimental.pallas import tpu_sc as plsc

NUM_CORES = 2
NUM_SUBCORES = 16
NUM_WORKERS = NUM_CORES * NUM_SUBCORES
LANES = 16
LUT_N = 512
ROWS = 64
NSLOT = 3
VPI = 2
UNROLL = 8


def _build_lut(control_points, u, dtype):
    t = jnp.linspace(-1.0, 1.0, LUT_N, dtype=dtype)
    cpy = control_points[:, 1]
    n = control_points.shape[0] - 1
    bern = jnp.stack(
        [comb(n, k) * t**k * (1.0 - t) ** (n - k) for k in range(n + 1)], axis=0
    )
    fp = jnp.clip(cpy @ bern, -1.0, 1.0)
    slope = (fp[1:] - fp[:-1]) / (t[1:] - t[:-1])
    a_seg = fp[:-1] - slope * t[:-1]
    a = jnp.concatenate([a_seg[:1], a_seg])
    b = jnp.concatenate([slope[:1], slope])
    sign = jnp.where(u[0] > 0.5, 1.0, -1.0).astype(dtype)
    return a * sign, b * sign


def _sc_body(img_hbm, a_hbm, b_hbm, out_hbm, xbuf, abuf, bbuf, isems, osems):
    n_imgs, _, height, width = img_hbm.shape
    imgs_per_w = n_imgs // NUM_WORKERS
    nchunk = height // ROWS
    chunk_px = ROWS * width

    cid = lax.axis_index("c")
    sid = lax.axis_index("s")
    wid = sid * NUM_CORES + cid

    pltpu.sync_copy(a_hbm, abuf)
    pltpu.sync_copy(b_hbm, bbuf)

    def interp(x):
        idx = jnp.clip((x * 255.5 + 256.5).astype(jnp.int32), 1, LUT_N - 1)
        av = plsc.load_gather(abuf, [idx])
        bv = plsc.load_gather(bbuf, [idx])
        return av + bv * x

    def start_in(img, c, s):
        pltpu.async_copy(
            img_hbm.at[img, 0, pl.ds(c * ROWS, ROWS)],
            xbuf.at[pl.ds(s * ROWS, ROWS)],
            isems.at[s],
        )

    def wait_in(img, s):
        pltpu.make_async_copy(
            img_hbm.at[img, 0, pl.ds(0, ROWS)],
            xbuf.at[pl.ds(s * ROWS, ROWS)],
            isems.at[s],
        ).wait()

    def start_out(img, c, s):
        pltpu.async_copy(
            xbuf.at[pl.ds(s * ROWS, ROWS)],
            out_hbm.at[img, 0, pl.ds(c * ROWS, ROWS)],
            osems.at[s],
        )

    def wait_out(img, s):
        pltpu.make_async_copy(
            xbuf.at[pl.ds(s * ROWS, ROWS)],
            out_hbm.at[img, 0, pl.ds(0, ROWS)],
            osems.at[s],
        ).wait()

    for im in range(imgs_per_w):
        img = wid * imgs_per_w + im

        start_in(img, 0, 0)
        acc0 = (
            jnp.full((LANES,), jnp.inf, jnp.float32),
            jnp.full((LANES,), -jnp.inf, jnp.float32),
        ) * VPI

        def chunk1(c, acc):
            s = c % 2
            row0 = s * ROWS

            @pl.when(c + 1 < nchunk)
            def _():
                start_in(img, c + 1, 1 - s)

            wait_in(img, s)

            def step1(i, carry):
                row = row0 + lax.shift_right_logical(i, 9)
                col = lax.bitwise_and(i, width - 1)
                out = []
                for k in range(VPI):
                    cmn, cmx = carry[2 * k], carry[2 * k + 1]
                    v = interp(xbuf[row, pl.ds(col + k * LANES, LANES)])
                    out += [jnp.minimum(cmn, v), jnp.maximum(cmx, v)]
                return tuple(out)

            return plsc.parallel_loop(
                0, chunk_px, VPI * LANES, unroll=UNROLL, carry=acc
            )(step1)

        acc = lax.fori_loop(0, nchunk, chunk1, acc0)
        mn = jnp.minimum(acc[0], *[acc[2 * k] for k in range(1, VPI)])
        mx = jnp.maximum(acc[1], *[acc[2 * k + 1] for k in range(1, VPI)])
        mnv = jnp.full((LANES,), jnp.min(mn), jnp.float32)
        mxv = jnp.full((LANES,), jnp.max(mx), jnp.float32)
        scv = 2.0 / (mxv - mnv)
        ofv = -mnv * scv - 1.0

        start_in(img, 0, 0)

        def chunk2(c, carry):
            s = c % NSLOT
            row0 = s * ROWS

            @pl.when(c + 1 < nchunk)
            def _():
                ns = (c + 1) % NSLOT

                @pl.when(c + 1 >= NSLOT)
                def _():
                    wait_out(img, ns)

                start_in(img, c + 1, ns)

            wait_in(img, s)

            def step2(i, icarry):
                row = row0 + lax.shift_right_logical(i, 9)
                col = lax.bitwise_and(i, width - 1)
                for k in range(VPI):
                    x = xbuf[row, pl.ds(col + k * LANES, LANES)]
                    xbuf[row, pl.ds(col + k * LANES, LANES)] = (
                        interp(x) * scv + ofv
                    )
                return icarry

            plsc.parallel_loop(
                0, chunk_px, VPI * LANES, unroll=UNROLL, carry=jnp.int32(0)
            )(step2)
            start_out(img, c, s)
            return carry

        lax.fori_loop(0, nchunk, chunk2, jnp.int32(0))
        for s in range(NSLOT):
            if s < nchunk:
                wait_out(img, s)


def kernel(image, control_points, u):
    a_lut, b_lut = _build_lut(control_points, u, image.dtype)

    mesh = plsc.VectorSubcoreMesh(
        core_axis_name="c",
        subcore_axis_name="s",
        num_cores=NUM_CORES,
        num_subcores=NUM_SUBCORES,
    )
    return pl.kernel(
        _sc_body,
        out_type=jax.ShapeDtypeStruct(image.shape, jnp.float32),
        mesh=mesh,
        compiler_params=pltpu.CompilerParams(needs_layout_passes=False),
        scratch_types=[
            pltpu.VMEM((NSLOT * ROWS, 512), jnp.float32),
            pltpu.VMEM((LUT_N,), jnp.float32),
            pltpu.VMEM((LUT_N,), jnp.float32),
            pltpu.SemaphoreType.DMA((NSLOT,)),
            pltpu.SemaphoreType.DMA((NSLOT,)),
        ],
    )(image, a_lut, b_lut)
